# baseline (speedup 1.0000x reference)
import functools

import jax
import jax.numpy as jnp
from jax import lax
from jax.experimental import pallas as pl
from jax.experimental.pallas import tpu as pltpu
from jax.experimental.pallas import tpu_sc as plsc

NC = 2
NS = 16
NW = NC * NS
LN_EPS = 1e-5


def _rsqrt16(t):
    i = plsc.bitcast(t, jnp.int32)
    i = jnp.int32(0x5F3759DF) - lax.shift_right_logical(i, 1)
    y = plsc.bitcast(i, jnp.float32)
    ht = t * 0.5
    for _ in range(3):
        y = y * (1.5 - ht * y * y)
    return y


@functools.partial(jax.jit, static_argnums=(5, 6, 7))
def _sc_embed_ln(xtf, embp, posf, gamma, beta, B, S, D):
    BW = B // NW
    NG = BW // 16
    NB = 4
    NBS = 2
    DT = D // 8

    mesh = plsc.VectorSubcoreMesh(core_axis_name="c", subcore_axis_name="s",
                                  num_cores=NC, num_subcores=NS)

    @functools.partial(
        pl.kernel,
        out_type=jax.ShapeDtypeStruct((B, S, 128), jnp.float32),
        mesh=mesh,
        compiler_params=pltpu.CompilerParams(needs_layout_passes=False,
                                             use_tc_tiling_on_sc=False),
        scratch_types=[
            pltpu.VMEM((S * D,), jnp.float32),
            pltpu.VMEM((D,), jnp.float32),
            pltpu.VMEM((D,), jnp.float32),
            [pltpu.VMEM((BW,), jnp.int32) for _ in range(NB)],
            [pltpu.VMEM((BW, D), jnp.float32) for _ in range(NB)],
            [pltpu.VMEM((BW, 128), jnp.float32) for _ in range(NBS)],
            [pltpu.SemaphoreType.DMA for _ in range(NB)],
            [pltpu.SemaphoreType.DMA for _ in range(NB)],
            [pltpu.SemaphoreType.DMA for _ in range(NBS)],
        ],
    )
    def k(x_hbm, emb_hbm, posf_hbm, gamma_hbm, beta_hbm, out_hbm,
          posb_v, gam_v, bet_v, ibufs, pbufs, obufs,
          isems, gsems, ssems):
        wid = lax.axis_index("s") * NC + lax.axis_index("c")
        b0 = wid * BW

        pltpu.sync_copy(posf_hbm, posb_v)
        pltpu.sync_copy(gamma_hbm, gam_v)
        pltpu.sync_copy(beta_hbm, bet_v)

        def fold(p, _):
            for kk in range(D // 16):
                posb_v[pl.ds(D * p + 16 * kk, 16)] = (
                    posb_v[pl.ds(D * p + 16 * kk, 16)]
                    + bet_v[pl.ds(16 * kk, 16)])
            return 0
        lax.fori_loop(0, S, fold, 0)

        def i_src(c):
            return x_hbm.at[pl.ds(c * B + b0, BW)]

        def start_idx(c, b):
            pltpu.async_copy(i_src(c), ibufs[b], isems[b])

        def wait_idx(c, b):
            pltpu.make_async_copy(i_src(c), ibufs[b], isems[b]).wait()

        def prep_gather(c, b):
            pltpu.async_copy(emb_hbm.at[ibufs[b]], pbufs[b], gsems[b])

        def wait_gather(c, b):
            pltpu.make_async_copy(emb_hbm.at[ibufs[b]], pbufs[b],
                                  gsems[b]).wait()

        def s_dst(c):
            return out_hbm.at[pl.ds(b0, BW), c]

        def start_store(c, b):
            pltpu.async_copy(obufs[b], s_dst(c), ssems[b])

        def wait_store(c, b):
            pltpu.make_async_copy(obufs[b], s_dst(c), ssems[b]).wait()

        def compute(c, gb, ob):
            pb = pbufs[gb]
            obuf = obufs[ob]
            NV = D // 16
            gam, pv = [], []
            for kk in range(NV):
                gam.append(gam_v[pl.ds(16 * kk, 16)])
                pv.append(posb_v[pl.ds(c * D + 16 * kk, 16)])

            @plsc.parallel_loop(0, BW, unroll=8)
            def t_body(t):
                v = [pb[t, pl.ds(16 * kk, 16)] for kk in range(NV)]
                s = (v[0] + v[1]) + (v[2] + v[3])
                q = ((v[0] * v[0] + v[1] * v[1])
                     + (v[2] * v[2] + v[3] * v[3]))
                mu = jnp.full((16,), jnp.sum(s) * (1.0 / D), jnp.float32)
                ex2 = jnp.full((16,), jnp.sum(q) * (1.0 / D), jnp.float32)
                var = ex2 - mu * mu
                rstd = _rsqrt16(var + LN_EPS)
                for kk in range(NV):
                    obuf[t, pl.ds(16 * kk, 16)] = (
                        (v[kk] - mu) * (rstd * gam[kk]) + pv[kk])

        start_idx(0, 0)
        start_idx(1, 1)
        start_idx(2, 2)
        wait_idx(0, 0)
        prep_gather(0, 0)
        wait_idx(1, 1)
        prep_gather(1, 1)

        def c_body(cb, _):
            for u in range(NB):
                c = cb + u
                gb = u
                ob = u % NBS

                @pl.when(c + 3 < S)
                def _():
                    start_idx(c + 3, (u + 3) % NB)

                @pl.when(c + 2 < S)
                def _():
                    wait_idx(c + 2, (u + 2) % NB)
                    prep_gather(c + 2, (u + 2) % NB)

                @pl.when(c >= NBS)
                def _():
                    wait_store(c - NBS, ob)

                wait_gather(c, gb)
                compute(c, gb, ob)
                start_store(c, ob)
            return 0

        lax.fori_loop(0, S // NB, lambda i, u: c_body(i * NB, u), 0)

        for c in range(S - NBS, S):
            wait_store(c, c % NBS)

    return k(xtf, embp, posf, gamma, beta)


def kernel(x, emb_table, pos_table, gamma, beta):
    B, S = x.shape
    D = emb_table.shape[1]
    xtf = x.T.reshape(S * B).astype(jnp.int32)
    posf = lax.slice_in_dim(pos_table, 1, S + 1, axis=0).reshape(S * D)
    out = _sc_embed_ln(xtf, emb_table, posf, gamma, beta, B, S, D)
    return lax.slice(out, (0, 0, 0), (B, S, D))


# speedup vs baseline: 1.2383x; 1.2383x over previous
import functools

import jax
import jax.numpy as jnp
from jax import lax
from jax.experimental import pallas as pl
from jax.experimental.pallas import tpu as pltpu
from jax.experimental.pallas import tpu_sc as plsc

NC = 2
NS = 16
NW = NC * NS
LN_EPS = 1e-5


def _rsqrt16(t):
    i = plsc.bitcast(t, jnp.int32)
    i = jnp.int32(0x5F3759DF) - lax.shift_right_logical(i, 1)
    y = plsc.bitcast(i, jnp.float32)
    ht = t * 0.5
    for _ in range(2):
        y = y * (1.5 - ht * y * y)
    return y


@functools.partial(jax.jit, static_argnums=(5, 6, 7))
def _sc_embed_ln(xtf, embp, posf, gamma, beta, B, S, D):
    BW = B // NW
    NG = BW // 16
    NB = 4
    NBS = 2
    DT = D // 8

    mesh = plsc.VectorSubcoreMesh(core_axis_name="c", subcore_axis_name="s",
                                  num_cores=NC, num_subcores=NS)

    @functools.partial(
        pl.kernel,
        out_type=jax.ShapeDtypeStruct((B, S, 128), jnp.float32),
        mesh=mesh,
        compiler_params=pltpu.CompilerParams(needs_layout_passes=False,
                                             use_tc_tiling_on_sc=False),
        scratch_types=[
            pltpu.VMEM((S * D,), jnp.float32),
            pltpu.VMEM((D,), jnp.float32),
            pltpu.VMEM((D,), jnp.float32),
            [pltpu.VMEM((BW,), jnp.int32) for _ in range(NB)],
            [pltpu.VMEM((BW, D), jnp.float32) for _ in range(NB)],
            [pltpu.VMEM((BW, 128), jnp.float32) for _ in range(NBS)],
            [pltpu.SemaphoreType.DMA for _ in range(NB)],
            [pltpu.SemaphoreType.DMA for _ in range(NB)],
            [pltpu.SemaphoreType.DMA for _ in range(NBS)],
        ],
    )
    def k(x_hbm, emb_hbm, posf_hbm, gamma_hbm, beta_hbm, out_hbm,
          posb_v, gam_v, bet_v, ibufs, pbufs, obufs,
          isems, gsems, ssems):
        wid = lax.axis_index("s") * NC + lax.axis_index("c")
        b0 = wid * BW

        pltpu.sync_copy(posf_hbm, posb_v)
        pltpu.sync_copy(gamma_hbm, gam_v)
        pltpu.sync_copy(beta_hbm, bet_v)

        def fold(p, _):
            for kk in range(D // 16):
                posb_v[pl.ds(D * p + 16 * kk, 16)] = (
                    posb_v[pl.ds(D * p + 16 * kk, 16)]
                    + bet_v[pl.ds(16 * kk, 16)])
            return 0
        lax.fori_loop(0, S, fold, 0)

        def i_src(c):
            return x_hbm.at[pl.ds(c * B + b0, BW)]

        def start_idx(c, b):
            pltpu.async_copy(i_src(c), ibufs[b], isems[b])

        def wait_idx(c, b):
            pltpu.make_async_copy(i_src(c), ibufs[b], isems[b]).wait()

        def prep_gather(c, b):
            pltpu.async_copy(emb_hbm.at[ibufs[b]], pbufs[b], gsems[b])

        def wait_gather(c, b):
            pltpu.make_async_copy(emb_hbm.at[ibufs[b]], pbufs[b],
                                  gsems[b]).wait()

        def s_dst(c):
            return out_hbm.at[pl.ds(b0, BW), c]

        def start_store(c, b):
            pltpu.async_copy(obufs[b], s_dst(c), ssems[b])

        def wait_store(c, b):
            pltpu.make_async_copy(obufs[b], s_dst(c), ssems[b]).wait()

        def compute(c, gb, ob):
            pb = pbufs[gb]
            obuf = obufs[ob]
            NV = D // 16
            gam, pv = [], []
            for kk in range(NV):
                gam.append(gam_v[pl.ds(16 * kk, 16)])
                pv.append(posb_v[pl.ds(c * D + 16 * kk, 16)])

            @plsc.parallel_loop(0, BW, unroll=4)
            def t_body(t):
                v = [pb[t, pl.ds(16 * kk, 16)] for kk in range(NV)]
                s = (v[0] + v[1]) + (v[2] + v[3])
                q = ((v[0] * v[0] + v[1] * v[1])
                     + (v[2] * v[2] + v[3] * v[3]))
                mu = jnp.full((16,), jnp.sum(s) * (1.0 / D), jnp.float32)
                ex2 = jnp.full((16,), jnp.sum(q) * (1.0 / D), jnp.float32)
                var = ex2 - mu * mu
                rstd = _rsqrt16(var + LN_EPS)
                for kk in range(NV):
                    obuf[t, pl.ds(16 * kk, 16)] = (
                        (v[kk] - mu) * (rstd * gam[kk]) + pv[kk])

        start_idx(0, 0)
        start_idx(1, 1)
        start_idx(2, 2)
        wait_idx(0, 0)
        prep_gather(0, 0)
        wait_idx(1, 1)
        prep_gather(1, 1)

        def c_body(cb, _):
            for u in range(NB):
                c = cb + u
                gb = u
                ob = u % NBS

                @pl.when(c + 3 < S)
                def _():
                    start_idx(c + 3, (u + 3) % NB)

                @pl.when(c + 2 < S)
                def _():
                    wait_idx(c + 2, (u + 2) % NB)
                    prep_gather(c + 2, (u + 2) % NB)

                @pl.when(c >= NBS)
                def _():
                    wait_store(c - NBS, ob)

                wait_gather(c, gb)
                compute(c, gb, ob)
                start_store(c, ob)
            return 0

        lax.fori_loop(0, S // NB, lambda i, u: c_body(i * NB, u), 0)

        for c in range(S - NBS, S):
            wait_store(c, c % NBS)

    return k(xtf, embp, posf, gamma, beta)


def kernel(x, emb_table, pos_table, gamma, beta):
    B, S = x.shape
    D = emb_table.shape[1]
    xtf = x.T.reshape(S * B).astype(jnp.int32)
    posf = lax.slice_in_dim(pos_table, 1, S + 1, axis=0).reshape(S * D)
    out = _sc_embed_ln(xtf, emb_table, posf, gamma, beta, B, S, D)
    return lax.slice(out, (0, 0, 0), (B, S, D))
